# baseline (speedup 1.0000x reference)
import functools

import jax
import jax.numpy as jnp
from jax import lax
from jax.experimental import pallas as pl
from jax.experimental.pallas import tpu as pltpu
from jax.experimental.pallas import tpu_sc as plsc

_LANES = 16
_LN2 = 0.6931471805599453
_LOG_COEFFS = (
    0.01000929,
    -0.052437536,
    0.13083343,
    -0.22316587,
    0.32722571,
    -0.49928504,
    0.9999671,
    2.554673e-07,
)


def _log_f32(x):
    bits = lax.bitcast_convert_type(x, jnp.int32)
    e = lax.shift_right_logical(bits, 23) - 127
    mbits = lax.bitwise_or(lax.bitwise_and(bits, 0x7FFFFF), 0x3F800000)
    m = lax.bitcast_convert_type(mbits, jnp.float32)
    u = m - jnp.float32(1.0)
    pol = jnp.full((_LANES,), _LOG_COEFFS[0], jnp.float32)
    for cf in _LOG_COEFFS[1:]:
        pol = pol * u + jnp.float32(cf)
    return e.astype(jnp.float32) * jnp.float32(_LN2) + pol


@functools.lru_cache(maxsize=8)
def _build_sc_loss(n: int, c: int, formula: bool):
    try:
        info = plsc.get_sparse_core_info()
        nc, ns = info.num_cores, info.num_subcores
    except Exception:
        nc, ns = 2, 16
    nw = nc * ns
    chunk = (n // (nw * _LANES)) * _LANES
    tail = n - nw * chunk
    nsub = 3 if chunk % 3 == 0 else 1
    sub = chunk // nsub
    tail_buf = max(tail, _LANES)
    unroll = 3
    piece = 656

    mesh = plsc.VectorSubcoreMesh(core_axis_name="c", subcore_axis_name="s")

    buf_types = []
    for size in (sub, sub):
        buf_types += [
            pltpu.VMEM((size,), jnp.int32),
            pltpu.VMEM((size,), jnp.int32),
            pltpu.VMEM((size,), jnp.float32),
            pltpu.VMEM((size,), jnp.float32),
        ]
    buf_types += [
        pltpu.VMEM((tail_buf,), jnp.int32),
        pltpu.VMEM((tail_buf,), jnp.int32),
        pltpu.VMEM((tail_buf,), jnp.float32),
        pltpu.VMEM((tail_buf,), jnp.float32),
        pltpu.VMEM((_LANES,), jnp.float32),
        pltpu.VMEM((2 * _LANES,), jnp.float32),
        pltpu.SemaphoreType.DMA,
        pltpu.SemaphoreType.DMA,
    ]

    @functools.partial(
        pl.kernel,
        out_type=jax.ShapeDtypeStruct((nw * _LANES,), jnp.float32),
        mesh=mesh,
        scratch_types=buf_types,
    )
    def sc_loss(yp_hbm, yt_hbm, cw_hbm, out_hbm,
                yt0, idx0, p0, w0, yt1, idx1, p1, w1,
                yt_t, idx_t, p_t, w_t, acc_v, ab_v, sem0, sem1):
        wid = lax.axis_index("s") * nc + lax.axis_index("c")
        iota = lax.broadcasted_iota(jnp.int32, (_LANES,), 0)
        cw_base = wid * c

        if formula:
            pltpu.sync_copy(cw_hbm, ab_v)
            av = ab_v[pl.ds(0, _LANES)]
            bv = ab_v[pl.ds(_LANES, _LANES)]

        def build(base, nvec, yt_ref, idx_ref, rewrite):
            def one(j):
                yt16 = yt_ref[pl.ds(j * _LANES, _LANES)]
                rows = (base + j * _LANES) + iota
                idx_ref[pl.ds(j * _LANES, _LANES)] = rows * c + yt16
                if rewrite:
                    yt_ref[pl.ds(j * _LANES, _LANES)] = yt16 + cw_base

            def ixb(jj, carry):
                for q in range(unroll):
                    one(jj * unroll + q)
                return carry
            lax.fori_loop(0, nvec // unroll, ixb, 0)
            for j in range(nvec - nvec % unroll, nvec):
                one(j)

        def fire(size, yt_ref, idx_ref, p_ref, w_ref, sem, with_w):
            copies = []
            off = 0
            while off < size:
                plen = min(piece, size - off)
                copies.append(pltpu.async_copy(
                    yp_hbm.at[idx_ref.at[pl.ds(off, plen)]],
                    p_ref.at[pl.ds(off, plen)], sem))
                if with_w:
                    copies.append(pltpu.async_copy(
                        cw_hbm.at[yt_ref.at[pl.ds(off, plen)]],
                        w_ref.at[pl.ds(off, plen)], sem))
                off += plen
            return copies

        def consume(nvec, yt_ref, p_ref, w_ref, acc_in, formula):
            def one(j, acc):
                p16 = p_ref[pl.ds(j * _LANES, _LANES)]
                if formula:
                    yt16 = yt_ref[pl.ds(j * _LANES, _LANES)]
                    w16 = av + bv * yt16.astype(jnp.float32)
                else:
                    w16 = w_ref[pl.ds(j * _LANES, _LANES)]
                return acc + w16 * _log_f32(p16 + jnp.float32(1e-7))

            def ab(jj, acc):
                for q in range(unroll):
                    acc = one(jj * unroll + q, acc)
                return acc
            acc = lax.fori_loop(0, nvec // unroll, ab, acc_in)
            for j in range(nvec - nvec % unroll, nvec):
                acc = one(j, acc)
            return acc

        def pipeline(formula):
            bufs = [(yt0, idx0, p0, w0, sem0), (yt1, idx1, p1, w1, sem1)]
            base = wid * chunk
            acc = jnp.zeros((_LANES,), jnp.float32)
            inflight = [None, None]
            for s in range(nsub):
                yt_ref, idx_ref, p_ref, w_ref, sem = bufs[s % 2]
                sb = base + s * sub
                pltpu.sync_copy(yt_hbm.at[pl.ds(sb, sub)], yt_ref)
                build(sb, sub // _LANES, yt_ref, idx_ref,
                      rewrite=not formula)
                inflight[s % 2] = fire(sub, yt_ref, idx_ref, p_ref, w_ref,
                                       sem, with_w=not formula)
                if s >= 1:
                    b = (s - 1) % 2
                    for cp in inflight[b]:
                        cp.wait()
                    inflight[b] = None
                    acc = consume(sub // _LANES, bufs[b][0], bufs[b][2],
                                  bufs[b][3], acc, formula)
            b = (nsub - 1) % 2
            for cp in inflight[b]:
                cp.wait()
            acc = consume(sub // _LANES, bufs[b][0], bufs[b][2], bufs[b][3],
                          acc, formula)
            acc_v[...] = acc

            if tail:
                @pl.when(wid == nw - 1)
                def _():
                    tb = nw * chunk
                    pltpu.sync_copy(yt_hbm.at[pl.ds(tb, tail)], yt_t)
                    build(tb, tail // _LANES, yt_t, idx_t,
                          rewrite=not formula)
                    for cp in fire(tail, yt_t, idx_t, p_t, w_t, sem0,
                                   with_w=not formula):
                        cp.wait()
                    tacc = consume(tail // _LANES, yt_t, p_t, w_t,
                                   jnp.zeros((_LANES,), jnp.float32),
                                   formula)
                    acc_v[...] = acc_v[...] + tacc

        pipeline(formula)

        pltpu.sync_copy(acc_v, out_hbm.at[pl.ds(wid * _LANES, _LANES)])

    return sc_loss, nw


def kernel(y_pred, y_true, class_weights):
    if y_pred.ndim == 3:
        y_pred = jnp.squeeze(y_pred, -1)
    n, c = y_pred.shape
    yp_flat = y_pred.reshape(-1)
    yt = y_true.reshape(-1).astype(jnp.int32)
    loss_formula, nw = _build_sc_loss(n, c, True)
    loss_gather, _ = _build_sc_loss(n, c, False)
    cw32 = class_weights.astype(jnp.float32)
    a = cw32[0]
    b = cw32[1] - cw32[0]
    fit = a + b * jnp.arange(c, dtype=jnp.float32)
    affine = jnp.max(jnp.abs(cw32 - fit)) < jnp.float32(1e-5)
    ab_vec = jnp.concatenate([
        jnp.full((_LANES,), a, jnp.float32),
        jnp.full((_LANES,), b, jnp.float32),
    ])
    cw_rep = jnp.tile(cw32, nw)
    partials = lax.cond(
        affine,
        lambda: loss_formula(yp_flat, yt, ab_vec),
        lambda: loss_gather(yp_flat, yt, cw_rep),
    )
    return -(jnp.sum(partials) / jnp.float32(n))
